# Initial kernel scaffold; baseline (speedup 1.0000x reference)
import functools

import jax
import jax.numpy as jnp
from jax.experimental import pallas as pl
from jax.experimental.pallas import tpu as pltpu

S = 1024


def _logits_body(x_ref, w1_ref, b1_ref, w2_ref, b2_ref, w_ref, gmax_ref):
    i = pl.program_id(0)
    h = jnp.tanh(
        jax.lax.dot(x_ref[...], w1_ref[...], preferred_element_type=jnp.float32)
        + b1_ref[...]
    )
    w = jnp.sum(h * w2_ref[...][None, :, 0], axis=1, keepdims=True) + b2_ref[0, 0]
    w_ref[...] = w
    bmax = jnp.max(w)

    @pl.when(i == 0)
    def _():
        gmax_ref[0, 0] = bmax

    @pl.when(i > 0)
    def _():
        gmax_ref[0, 0] = jnp.maximum(gmax_ref[0, 0], bmax)


def _pool_body(seg_ref, x_ref, w_ref, gmax_ref, out_ref, numer_ref, denom_ref,
               *, nseg, nblocks):
    i = pl.program_id(0)

    @pl.when(i == 0)
    def _():
        numer_ref[...] = jnp.zeros_like(numer_ref)
        denom_ref[...] = jnp.zeros_like(denom_ref)

    e = jnp.exp(w_ref[...] - gmax_ref[0, 0])
    y = x_ref[...] * e
    seg = seg_ref[0]
    iota = jax.lax.broadcasted_iota(jnp.int32, (nseg, seg.shape[1]), 0)
    onehot_t = (iota == seg).astype(jnp.float32)
    numer_ref[...] += jax.lax.dot(onehot_t, y, preferred_element_type=jnp.float32)
    denom_ref[...] += jax.lax.dot(onehot_t, e, preferred_element_type=jnp.float32)

    @pl.when(i == nblocks - 1)
    def _():
        d = denom_ref[...]
        out_ref[...] = jnp.where(d > 0.0, numer_ref[...] / d, 0.0)


def _pooling(x, batch, W1, b1, W2, b2, nseg, block):
    n, d = x.shape
    assert n % block == 0
    nblocks = n // block
    seg = batch.astype(jnp.int32).reshape(nblocks, 1, block)

    w, gmax = pl.pallas_call(
        _logits_body,
        grid=(nblocks,),
        in_specs=[
            pl.BlockSpec((block, d), lambda i: (i, 0)),
            pl.BlockSpec((d, d), lambda i: (0, 0)),
            pl.BlockSpec((d,), lambda i: (0,)),
            pl.BlockSpec((d, 1), lambda i: (0, 0)),
            pl.BlockSpec((1, 1), lambda i: (0, 0)),
        ],
        out_specs=[
            pl.BlockSpec((block, 1), lambda i: (i, 0)),
            pl.BlockSpec((1, 1), lambda i: (0, 0)),
        ],
        out_shape=[
            jax.ShapeDtypeStruct((n, 1), jnp.float32),
            jax.ShapeDtypeStruct((1, 1), jnp.float32),
        ],
    )(x, W1, b1, W2, b2.reshape(1, 1))

    out = pl.pallas_call(
        functools.partial(_pool_body, nseg=nseg, nblocks=nblocks),
        grid=(nblocks,),
        in_specs=[
            pl.BlockSpec((1, 1, block), lambda i: (i, 0, 0)),
            pl.BlockSpec((block, d), lambda i: (i, 0)),
            pl.BlockSpec((block, 1), lambda i: (i, 0)),
            pl.BlockSpec((1, 1), lambda i: (0, 0)),
        ],
        out_specs=pl.BlockSpec((nseg, d), lambda i: (0, 0)),
        out_shape=jax.ShapeDtypeStruct((nseg, d), jnp.float32),
        scratch_shapes=[
            pltpu.VMEM((nseg, d), jnp.float32),
            pltpu.VMEM((nseg, 1), jnp.float32),
        ],
    )(seg, x, w, gmax)
    return out


def kernel(x, batch, W1, b1, W2, b2):
    return _pooling(x, batch, W1, b1, W2, b2, nseg=S, block=2000)


# speedup vs baseline: 7.1219x; 7.1219x over previous
import functools

import jax
import jax.numpy as jnp
from jax.experimental import pallas as pl
from jax.experimental.pallas import tpu as pltpu

S = 1024


def _logits_body(x_ref, w1_ref, b1_ref, w2_ref, b2_ref, w_ref, gmax_ref):
    i = pl.program_id(0)
    h = jnp.tanh(
        jax.lax.dot(x_ref[...], w1_ref[...], preferred_element_type=jnp.float32)
        + b1_ref[...]
    )
    w = jnp.sum(h * w2_ref[...][None, :, 0], axis=1, keepdims=True) + b2_ref[...]
    w_ref[...] = w
    bmax = jnp.max(w, keepdims=True)

    @pl.when(i == 0)
    def _():
        gmax_ref[...] = bmax

    @pl.when(i > 0)
    def _():
        gmax_ref[...] = jnp.maximum(gmax_ref[...], bmax)


def _pool_body(seg_ref, x_ref, w_ref, gmax_ref, out_ref, numer_ref, denom_ref,
               *, nseg, nblocks):
    i = pl.program_id(0)

    @pl.when(i == 0)
    def _():
        numer_ref[...] = jnp.zeros_like(numer_ref)
        denom_ref[...] = jnp.zeros_like(denom_ref)

    e = jnp.exp(w_ref[...] - gmax_ref[...])
    y = x_ref[...] * e
    seg = seg_ref[0]
    iota = jax.lax.broadcasted_iota(jnp.int32, (nseg, seg.shape[1]), 0)
    onehot_t = (iota == seg).astype(jnp.float32)
    numer_ref[...] += jax.lax.dot(onehot_t, y, preferred_element_type=jnp.float32)
    denom_ref[...] += jax.lax.dot(onehot_t, e, preferred_element_type=jnp.float32)

    @pl.when(i == nblocks - 1)
    def _():
        d = denom_ref[...]
        out_ref[...] = jnp.where(d > 0.0, numer_ref[...] / d, 0.0)


def _pooling(x, batch, W1, b1, W2, b2, nseg, block):
    n, d = x.shape
    assert n % block == 0
    nblocks = n // block
    seg = batch.astype(jnp.int32).reshape(nblocks, 1, block)

    w, gmax = pl.pallas_call(
        _logits_body,
        grid=(nblocks,),
        in_specs=[
            pl.BlockSpec((block, d), lambda i: (i, 0)),
            pl.BlockSpec((d, d), lambda i: (0, 0)),
            pl.BlockSpec((d,), lambda i: (0,)),
            pl.BlockSpec((d, 1), lambda i: (0, 0)),
            pl.BlockSpec((1, 1), lambda i: (0, 0)),
        ],
        out_specs=[
            pl.BlockSpec((block, 1), lambda i: (i, 0)),
            pl.BlockSpec((1, 1), lambda i: (0, 0)),
        ],
        out_shape=[
            jax.ShapeDtypeStruct((n, 1), jnp.float32),
            jax.ShapeDtypeStruct((1, 1), jnp.float32),
        ],
    )(x, W1, b1, W2, b2.reshape(1, 1))

    out = pl.pallas_call(
        functools.partial(_pool_body, nseg=nseg, nblocks=nblocks),
        grid=(nblocks,),
        in_specs=[
            pl.BlockSpec((1, 1, block), lambda i: (i, 0, 0)),
            pl.BlockSpec((block, d), lambda i: (i, 0)),
            pl.BlockSpec((block, 1), lambda i: (i, 0)),
            pl.BlockSpec((1, 1), lambda i: (0, 0)),
        ],
        out_specs=pl.BlockSpec((nseg, d), lambda i: (0, 0)),
        out_shape=jax.ShapeDtypeStruct((nseg, d), jnp.float32),
        scratch_shapes=[
            pltpu.VMEM((nseg, d), jnp.float32),
            pltpu.VMEM((nseg, 1), jnp.float32),
        ],
    )(seg, x, w, gmax)
    return out


def kernel(x, batch, W1, b1, W2, b2):
    return _pooling(x, batch, W1, b1, W2, b2, nseg=S, block=2000)
